# baseline (speedup 1.0000x reference)
import functools

import jax
import jax.numpy as jnp
from jax import lax
from jax.experimental import pallas as pl
from jax.experimental.pallas import tpu as pltpu
from jax.experimental.pallas import tpu_sc as plsc


def _make_sc_gather(N, D, n_workers, chunk):
    b_per_w = N // n_workers
    n_chunks = b_per_w // chunk
    mesh = plsc.VectorSubcoreMesh(core_axis_name="c", subcore_axis_name="s")

    @functools.partial(
        pl.kernel,
        mesh=mesh,
        out_type=jax.ShapeDtypeStruct((N, D), jnp.float32),
        scratch_types=[
            pltpu.VMEM((chunk,), jnp.int32),
            pltpu.VMEM((chunk, D), jnp.float32),
            pltpu.SemaphoreType.DMA,
        ],
    )
    def k(table_hbm, idx_hbm, out_hbm, idx_v, rows_v, sem):
        wid = lax.axis_index("s") * 2 + lax.axis_index("c")
        base = wid * b_per_w

        def body(i, carry):
            off = base + i * chunk
            pltpu.sync_copy(idx_hbm.at[pl.ds(off, chunk)], idx_v)
            pltpu.async_copy(table_hbm.at[idx_v], rows_v, sem).wait()
            pltpu.sync_copy(rows_v, out_hbm.at[pl.ds(off, chunk)])
            return carry

        lax.fori_loop(0, n_chunks, body, 0)

    return k


def kernel(token_type_ids, embedding_weight):
    B, S = token_type_ids.shape
    V, D = embedding_weight.shape
    N = B * S
    ids = token_type_ids.reshape(N).astype(jnp.int32)
    out = _make_sc_gather(N, D, n_workers=32, chunk=32)(embedding_weight, ids)
    return out.reshape(B, S, D)


# speedup vs baseline: 5.6094x; 5.6094x over previous
import functools

import jax
import jax.numpy as jnp
from jax import lax
from jax.experimental import pallas as pl
from jax.experimental.pallas import tpu as pltpu
from jax.experimental.pallas import tpu_sc as plsc


def _make_sc_lookup(N, V, D, n_workers, n_inflight):
    b_per_w = N // n_workers
    mesh = plsc.VectorSubcoreMesh(core_axis_name="c", subcore_axis_name="s")

    @functools.partial(
        pl.kernel,
        mesh=mesh,
        out_type=jax.ShapeDtypeStruct((N, D), jnp.float32),
        scratch_types=[
            pltpu.VMEM((V, D), jnp.float32),
            pltpu.VMEM((b_per_w,), jnp.int32),
            pltpu.SemaphoreType.DMA,
        ],
    )
    def k(table_hbm, idx_hbm, out_hbm, table_v, idx_v, sem):
        wid = lax.axis_index("s") * 2 + lax.axis_index("c")
        base = wid * b_per_w

        pltpu.sync_copy(table_hbm, table_v)
        pltpu.sync_copy(idx_hbm.at[pl.ds(base, b_per_w)], idx_v)

        def drain_one():
            pltpu.make_async_copy(
                table_v.at[0], out_hbm.at[base], sem).wait()

        def body(g, carry):
            ids16 = idx_v[pl.ds(g * 16, 16)]
            for j in range(16):
                pltpu.async_copy(
                    table_v.at[ids16[j]],
                    out_hbm.at[base + g * 16 + j], sem)
            for _ in range(16):
                drain_one()
            return carry

        lax.fori_loop(0, b_per_w // 16, body, 0)

    return k


def kernel(token_type_ids, embedding_weight):
    B, S = token_type_ids.shape
    V, D = embedding_weight.shape
    N = B * S
    ids = token_type_ids.reshape(N).astype(jnp.int32)
    out = _make_sc_lookup(N, V, D, n_workers=32, n_inflight=8)(
        embedding_weight, ids)
    return out.reshape(B, S, D)
